# baseline (speedup 1.0000x reference)
import jax
import jax.numpy as jnp
from jax.experimental import pallas as pl
from jax.experimental.pallas import tpu as pltpu

_N, _D, _H = 10000, 128, 16
_BM = 200
_NB = _N // _BM


def _mega_kernel(adj_ref, x_ref, wcat_ref, wfc_ref, aff_ref,
                 mu_ref, lv_ref, xrec_ref, rec_ref,
                 xw_s, z_s, zt_s):
    s = pl.program_id(0)

    @pl.when(s == 0)
    def _xw_phase():
        xw_s[...] = jnp.dot(x_ref[...], wcat_ref[...],
                            preferred_element_type=jnp.float32)

    @pl.when((s >= 1) & (s <= _NB))
    def _gc_phase():
        t = jnp.dot(adj_ref[...], xw_s[...],
                    preferred_element_type=jnp.float32)
        t = jnp.where(t >= 0, t, 0.01 * t)
        mu = t[:, :_H]
        mu_ref[...] = mu
        lv_ref[...] = t[:, _H:]
        z_s[pl.ds((s - 1) * _BM, _BM), :] = mu
        h = jnp.dot(mu, wfc_ref[...], preferred_element_type=jnp.float32)
        xrec_ref[...] = h * aff_ref[0:1, :] + aff_ref[1:2, :]

    @pl.when(s > _NB)
    def _ip_phase():
        @pl.when(s == _NB + 1)
        def _build_zt():
            zt_s[...] = z_s[...].T

        zb = z_s[pl.ds((s - _NB - 1) * _BM, _BM), :]
        rec_ref[...] = jnp.dot(zb, zt_s[...],
                               preferred_element_type=jnp.float32)


def kernel(x, adj, W1, W2, Wfc, bfc, gamma, beta, running_mean, running_var):
    n, d = x.shape
    h = W1.shape[1]

    wcat = jnp.concatenate([W1, W2], axis=1)
    scale = gamma * jax.lax.rsqrt(running_var + 1e-5)
    shift = (bfc - running_mean) * scale + beta
    aff = jnp.stack([scale, shift], axis=0)

    gc_idx = lambda s: (jnp.clip(s - 1, 0, _NB - 1), 0)
    ip_idx = lambda s: (jnp.clip(s - _NB - 1, 0, _NB - 1), 0)

    mu, logvar, x_rec, adj_rec = pl.pallas_call(
        _mega_kernel,
        grid=(1 + 2 * _NB,),
        in_specs=[
            pl.BlockSpec((_BM, n), gc_idx),
            pl.BlockSpec((n, d), lambda s: (0, 0)),
            pl.BlockSpec((d, 2 * h), lambda s: (0, 0)),
            pl.BlockSpec((h, d), lambda s: (0, 0)),
            pl.BlockSpec((2, d), lambda s: (0, 0)),
        ],
        out_specs=[
            pl.BlockSpec((_BM, h), gc_idx),
            pl.BlockSpec((_BM, h), gc_idx),
            pl.BlockSpec((_BM, d), gc_idx),
            pl.BlockSpec((_BM, n), ip_idx),
        ],
        out_shape=[
            jax.ShapeDtypeStruct((n, h), jnp.float32),
            jax.ShapeDtypeStruct((n, h), jnp.float32),
            jax.ShapeDtypeStruct((n, d), jnp.float32),
            jax.ShapeDtypeStruct((n, n), jnp.float32),
        ],
        scratch_shapes=[
            pltpu.VMEM((n, 2 * h), jnp.float32),
            pltpu.VMEM((n, h), jnp.float32),
            pltpu.VMEM((h, n), jnp.float32),
        ],
    )(adj, x, wcat, Wfc, aff)

    z = mu
    return (adj_rec, mu, logvar, z, x_rec)
